# baseline (speedup 1.0000x reference)
import functools

import jax
import jax.numpy as jnp
from jax import lax
from jax.experimental import pallas as pl
from jax.experimental.pallas import tpu as pltpu
from jax.experimental.pallas import tpu_sc as plsc

N = 10000
E = 160000
D = 256
DH = D // 2
DE = 4
K = 4
NH = 40000
KNN = 3

NC = 2
NS = 16
NW = NC * NS

EC = 128
EN_CHUNKS = E // EC
EFULL = EN_CHUNKS // NW
EREM = EN_CHUNKS - EFULL * NW

RB = (N // NS) // 8 * 8
RTAIL = N - NS * RB

NHP = 40960
GC = 128
G_PER_W = NHP // NW
GN_CH = G_PER_W // GC

@functools.cache
def _mesh():
    return plsc.VectorSubcoreMesh(core_axis_name="c", subcore_axis_name="s",
                                  num_cores=NC, num_subcores=NS)


def _a_body(x_ref, wk_ref, al_ref, wr_ref, b_ref, y0_ref, y1_ref, t_ref):
    a = al_ref[0, :]
    a = a - jnp.max(a)
    ea = jnp.exp(a)
    a = ea / jnp.sum(ea)
    weff = a[0] * wk_ref[0]
    for k in range(1, K):
        weff = weff + a[k] * wk_ref[k]
    x = x_ref[...]
    y = jnp.dot(x, weff, preferred_element_type=jnp.float32)
    y0_ref[...] = y[:, :DH]
    y1_ref[...] = y[:, DH:]
    t_ref[...] = x + jnp.dot(x, wr_ref[...], preferred_element_type=jnp.float32) + b_ref[...]


def _run_a(x, wxx_k, alpha, wr, b):
    blk = 1000
    grid = N // blk
    return pl.pallas_call(
        _a_body,
        grid=(grid,),
        in_specs=[
            pl.BlockSpec((blk, D), lambda i: (i, 0)),
            pl.BlockSpec((K, D, D), lambda i: (0, 0, 0)),
            pl.BlockSpec((1, K), lambda i: (0, 0)),
            pl.BlockSpec((D, D), lambda i: (0, 0)),
            pl.BlockSpec((1, D), lambda i: (0, 0)),
        ],
        out_specs=[
            pl.BlockSpec((blk, DH), lambda i: (i, 0)),
            pl.BlockSpec((blk, DH), lambda i: (i, 0)),
            pl.BlockSpec((blk, D), lambda i: (i, 0)),
        ],
        out_shape=[
            jax.ShapeDtypeStruct((N, DH), jnp.float32),
            jax.ShapeDtypeStruct((N, DH), jnp.float32),
            jax.ShapeDtypeStruct((N, D), jnp.float32),
        ],
    )(x, wxx_k, alpha, wr, b)


def _b_body_ea(eap, dst, z128, ea_out, dst_v, ea_v, shared_ea):
    c = lax.axis_index("c")
    s = lax.axis_index("s")
    wid = s * NC + c

    @pl.when(s == 0)
    def _():
        pltpu.sync_copy(z128, shared_ea)

    plsc.subcore_barrier()

    def ea_round(base):
        pltpu.sync_copy(dst.at[pl.ds(base, EC)], dst_v)
        pltpu.sync_copy(eap.at[pl.ds(base, EC), :], ea_v)
        pltpu.sync_copy(ea_v, shared_ea.at[dst_v], add=True)

    for j in range(EFULL):
        ea_round((wid + j * NW) * EC)

    @pl.when(wid < EREM)
    def _():
        ea_round((wid + EFULL * NW) * EC)

    plsc.subcore_barrier()
    r0 = s * RB
    pltpu.sync_copy(shared_ea.at[pl.ds(r0, RB)], ea_out.at[c, pl.ds(r0, RB)])

    @pl.when(s == NS - 1)
    def _():
        pltpu.sync_copy(shared_ea.at[pl.ds(NS * RB, RTAIL)],
                        ea_out.at[c, pl.ds(NS * RB, RTAIL)])


def _b_body(yh, src, dst, z128, acc_out, src_v, dst_v, rows_v, shared, sem):
    c = lax.axis_index("c")
    s = lax.axis_index("s")
    wid = s * NC + c

    @pl.when(s == 0)
    def _():
        pltpu.sync_copy(z128, shared)

    plsc.subcore_barrier()

    def edge_round(base):
        pltpu.sync_copy(src.at[pl.ds(base, EC)], src_v)
        pltpu.sync_copy(dst.at[pl.ds(base, EC)], dst_v)
        pltpu.async_copy(yh.at[src_v], rows_v, sem).wait()
        pltpu.sync_copy(rows_v, shared.at[dst_v], add=True)

    for j in range(EFULL):
        edge_round((wid + j * NW) * EC)

    @pl.when(wid < EREM)
    def _():
        edge_round((wid + EFULL * NW) * EC)

    plsc.subcore_barrier()
    r0 = s * RB
    pltpu.sync_copy(shared.at[pl.ds(r0, RB)], acc_out.at[c, pl.ds(r0, RB)])

    @pl.when(s == NS - 1)
    def _():
        pltpu.sync_copy(shared.at[pl.ds(NS * RB, RTAIL)],
                        acc_out.at[c, pl.ds(NS * RB, RTAIL)])


@functools.cache
def _scatter_ea():
    return pl.kernel(
        _b_body_ea,
        out_type=jax.ShapeDtypeStruct((NC, N, DH), jnp.float32),
        mesh=_mesh(),
        scratch_types=[
            pltpu.VMEM((EC,), jnp.int32),
            pltpu.VMEM((EC, DH), jnp.float32),
            pltpu.VMEM_SHARED((N, DH), jnp.float32),
        ],
    )


@functools.cache
def _scatter():
    return pl.kernel(
        _b_body,
        out_type=jax.ShapeDtypeStruct((NC, N, DH), jnp.float32),
        mesh=_mesh(),
        scratch_types=[
            pltpu.VMEM((EC,), jnp.int32),
            pltpu.VMEM((EC,), jnp.int32),
            pltpu.VMEM((EC, DH), jnp.float32),
            pltpu.VMEM_SHARED((N, DH), jnp.float32),
            pltpu.SemaphoreType.DMA,
        ],
    )


def _c_body(t_ref, a0_ref, a1_ref, eaa_ref, wea_ref, al_ref, f_ref):
    a = al_ref[0, :]
    a = a - jnp.max(a)
    e = jnp.exp(a)
    a = e / jnp.sum(e)
    wea = a[0] * wea_ref[0]
    for k in range(1, K):
        wea = wea + a[k] * wea_ref[k]
    agg = jnp.concatenate([a0_ref[0] + a0_ref[1], a1_ref[0] + a1_ref[1]], axis=1)
    eat = (eaa_ref[0] + eaa_ref[1])[:, :DE]
    f_ref[...] = (t_ref[...] + agg
                  + jnp.dot(eat, wea, preferred_element_type=jnp.float32))


def _run_c(t, acc0, acc1, ea_acc, wea_k, alpha):
    blk = 1000
    grid = N // blk
    return pl.pallas_call(
        _c_body,
        grid=(grid,),
        in_specs=[
            pl.BlockSpec((blk, D), lambda i: (i, 0)),
            pl.BlockSpec((NC, blk, DH), lambda i: (0, i, 0)),
            pl.BlockSpec((NC, blk, DH), lambda i: (0, i, 0)),
            pl.BlockSpec((NC, blk, DH), lambda i: (0, i, 0)),
            pl.BlockSpec((K, DE, D), lambda i: (0, 0, 0)),
            pl.BlockSpec((1, K), lambda i: (0, 0)),
        ],
        out_specs=pl.BlockSpec((blk, D), lambda i: (i, 0)),
        out_shape=jax.ShapeDtypeStruct((N, D), jnp.float32),
    )(t, acc0, acc1, ea_acc, wea_k, alpha)


def _d_body(ph_ref, pt_ref, idx_ref, wn_ref):
    ph = ph_ref[...]
    pt = pt_ref[...]
    pn = pt[0:1, :] * pt[0:1, :] + pt[1:2, :] * pt[1:2, :] + pt[2:3, :] * pt[2:3, :]
    s = pn - 2.0 * jnp.dot(ph, pt, preferred_element_type=jnp.float32,
                           precision=lax.Precision.HIGHEST)
    hn = (ph[:, 0:1] * ph[:, 0:1] + ph[:, 1:2] * ph[:, 1:2]
          + ph[:, 2:3] * ph[:, 2:3])
    cid = lax.broadcasted_iota(jnp.int32, s.shape, 1)
    ids = []
    ws = []
    for _ in range(KNN):
        m = jnp.min(s, axis=1, keepdims=True)
        am = jnp.min(jnp.where(s == m, cid, N), axis=1, keepdims=True)
        ids.append(am)
        d2 = jnp.maximum(m + hn, 0.0)
        ws.append(1.0 / (d2 + 1e-16))
        s = jnp.where(cid == am, 3.0e38, s)
    w = jnp.concatenate(ws, axis=1)
    idx_ref[...] = jnp.concatenate(ids, axis=1)
    wn_ref[...] = w / jnp.sum(w, axis=1, keepdims=True)


def _run_d(pos_high, pos_t):
    blk = 200
    grid = NH // blk
    return pl.pallas_call(
        _d_body,
        grid=(grid,),
        in_specs=[
            pl.BlockSpec((blk, 3), lambda i: (i, 0)),
            pl.BlockSpec((3, N), lambda i: (0, 0)),
        ],
        out_specs=[
            pl.BlockSpec((blk, KNN), lambda i: (i, 0)),
            pl.BlockSpec((blk, KNN), lambda i: (i, 0)),
        ],
        out_shape=[
            jax.ShapeDtypeStruct((NH, KNN), jnp.int32),
            jax.ShapeDtypeStruct((NH, KNN), jnp.float32),
        ],
    )(pos_high, pos_t)


def _e_body(f, i0, i1, i2, g0, g1, g2, idx_v, rows_v, sem):
    c = lax.axis_index("c")
    s = lax.axis_index("s")
    wid = s * NC + c
    for ih, gh in ((i0, g0), (i1, g1), (i2, g2)):
        for j in range(GN_CH):
            base = wid * G_PER_W + j * GC
            pltpu.sync_copy(ih.at[pl.ds(base, GC)], idx_v)
            pltpu.async_copy(f.at[idx_v], rows_v, sem).wait()
            pltpu.sync_copy(rows_v, gh.at[pl.ds(base, GC)])


@functools.cache
def _gather3():
    return pl.kernel(
        _e_body,
        out_type=(
            jax.ShapeDtypeStruct((NHP, D), jnp.float32),
            jax.ShapeDtypeStruct((NHP, D), jnp.float32),
            jax.ShapeDtypeStruct((NHP, D), jnp.float32),
        ),
        mesh=_mesh(),
        scratch_types=[
            pltpu.VMEM((GC,), jnp.int32),
            pltpu.VMEM((GC, D), jnp.float32),
            pltpu.SemaphoreType.DMA,
        ],
    )


def _f_body(g0_ref, g1_ref, g2_ref, wn_ref, o_ref):
    wn = wn_ref[...]
    o_ref[...] = (g0_ref[...] * wn[:, 0:1] + g1_ref[...] * wn[:, 1:2]
                  + g2_ref[...] * wn[:, 2:3])


def _run_f(g0, g1, g2, wn):
    blk = 200
    grid = NH // blk
    return pl.pallas_call(
        _f_body,
        grid=(grid,),
        in_specs=[
            pl.BlockSpec((blk, D), lambda i: (i, 0)),
            pl.BlockSpec((blk, D), lambda i: (i, 0)),
            pl.BlockSpec((blk, D), lambda i: (i, 0)),
            pl.BlockSpec((blk, KNN), lambda i: (i, 0)),
        ],
        out_specs=pl.BlockSpec((blk, D), lambda i: (i, 0)),
        out_shape=jax.ShapeDtypeStruct((NH, D), jnp.float32),
    )(g0, g1, g2, wn)


@jax.jit
def _kernel_impl(x, edge_index, edge_attr, pos, pos_high, W2, alpha2, Wr2, b2):
    wxx_k = W2[:, :D, :]
    wea_k = W2[:, D:, :]
    alpha = alpha2.reshape(1, K)
    brow = b2.reshape(1, D)
    src = edge_index[0]
    dst = edge_index[1]
    eap = jnp.pad(edge_attr, ((0, 0), (0, DH - DE)))
    z128 = jnp.zeros((N, DH), jnp.float32)

    y0, y1, t = _run_a(x, wxx_k, alpha, Wr2, brow)
    scatter = _scatter()
    acc0 = scatter(y0, src, dst, z128)
    acc1 = scatter(y1, src, dst, z128)
    ea_acc = _scatter_ea()(eap, dst, z128)
    f = _run_c(t, acc0, acc1, ea_acc, wea_k, alpha)

    pos_t = pos.T
    idx, wn = _run_d(pos_high, pos_t)
    ipad = jnp.zeros((KNN, NHP), jnp.int32).at[:, :NH].set(idx.T)
    g0, g1, g2 = _gather3()(f, ipad[0], ipad[1], ipad[2])
    return _run_f(g0, g1, g2, wn)


def kernel(x, edge_index, edge_attr, pos, pos_high, W1, alpha1, Wr1, b1,
           W2, alpha2, Wr2, b2):
    return _kernel_impl(x, edge_index, edge_attr, pos, pos_high,
                        W2, alpha2, Wr2, b2)


# speedup vs baseline: 1.3757x; 1.3757x over previous
import functools

import jax
import jax.numpy as jnp
from jax import lax
from jax.experimental import pallas as pl
from jax.experimental.pallas import tpu as pltpu
from jax.experimental.pallas import tpu_sc as plsc

N = 10000
E = 160000
D = 256
DH = D // 2
DE = 4
K = 4
NH = 40000
KNN = 3

NC = 2
NS = 16
NW = NC * NS

EC = 128
EN_CHUNKS = E // EC
EFULL = EN_CHUNKS // NW
EREM = EN_CHUNKS - EFULL * NW

RB = (N // NS) // 8 * 8
RTAIL = N - NS * RB

NHP = 40960
GC = 128
G_PER_W = NHP // NW
GN_CH = G_PER_W // GC

@functools.cache
def _mesh():
    return plsc.VectorSubcoreMesh(core_axis_name="c", subcore_axis_name="s",
                                  num_cores=NC, num_subcores=NS)


def _a_body(x_ref, wk_ref, al_ref, wr_ref, b_ref, y0_ref, y1_ref, t_ref):
    a = al_ref[0, :]
    a = a - jnp.max(a)
    ea = jnp.exp(a)
    a = ea / jnp.sum(ea)
    weff = a[0] * wk_ref[0]
    for k in range(1, K):
        weff = weff + a[k] * wk_ref[k]
    x = x_ref[...]
    y = jnp.dot(x, weff, preferred_element_type=jnp.float32)
    y0_ref[...] = y[:, :DH]
    y1_ref[...] = y[:, DH:]
    t_ref[...] = x + jnp.dot(x, wr_ref[...], preferred_element_type=jnp.float32) + b_ref[...]


def _run_a(x, wxx_k, alpha, wr, b):
    blk = 1000
    grid = N // blk
    return pl.pallas_call(
        _a_body,
        grid=(grid,),
        in_specs=[
            pl.BlockSpec((blk, D), lambda i: (i, 0)),
            pl.BlockSpec((K, D, D), lambda i: (0, 0, 0)),
            pl.BlockSpec((1, K), lambda i: (0, 0)),
            pl.BlockSpec((D, D), lambda i: (0, 0)),
            pl.BlockSpec((1, D), lambda i: (0, 0)),
        ],
        out_specs=[
            pl.BlockSpec((blk, DH), lambda i: (i, 0)),
            pl.BlockSpec((blk, DH), lambda i: (i, 0)),
            pl.BlockSpec((blk, D), lambda i: (i, 0)),
        ],
        out_shape=[
            jax.ShapeDtypeStruct((N, DH), jnp.float32),
            jax.ShapeDtypeStruct((N, DH), jnp.float32),
            jax.ShapeDtypeStruct((N, D), jnp.float32),
        ],
    )(x, wxx_k, alpha, wr, b)


def _b_body_ea(eap, dst, z128, ea_out, dst_v, ea_v, shared_ea):
    c = lax.axis_index("c")
    s = lax.axis_index("s")
    wid = s * NC + c

    @pl.when(s == 0)
    def _():
        pltpu.sync_copy(z128, shared_ea)

    plsc.subcore_barrier()

    def ea_round(base):
        pltpu.sync_copy(dst.at[pl.ds(base, EC)], dst_v)
        pltpu.sync_copy(eap.at[pl.ds(base, EC), :], ea_v)
        pltpu.sync_copy(ea_v, shared_ea.at[dst_v], add=True)

    for j in range(EFULL):
        ea_round((wid + j * NW) * EC)

    @pl.when(wid < EREM)
    def _():
        ea_round((wid + EFULL * NW) * EC)

    plsc.subcore_barrier()
    r0 = s * RB
    pltpu.sync_copy(shared_ea.at[pl.ds(r0, RB)], ea_out.at[c, pl.ds(r0, RB)])

    @pl.when(s == NS - 1)
    def _():
        pltpu.sync_copy(shared_ea.at[pl.ds(NS * RB, RTAIL)],
                        ea_out.at[c, pl.ds(NS * RB, RTAIL)])


def _b_body(yh, src, dst, z128, acc_out,
            src_v0, dst_v0, rows_v0, src_v1, dst_v1, rows_v1,
            shared, gsem0, gsem1, ssem0, ssem1):
    c = lax.axis_index("c")
    s = lax.axis_index("s")
    wid = s * NC + c

    @pl.when(s == 0)
    def _():
        pltpu.sync_copy(z128, shared)

    plsc.subcore_barrier()

    srcs = (src_v0, src_v1)
    dsts = (dst_v0, dst_v1)
    rows = (rows_v0, rows_v1)
    gsems = (gsem0, gsem1)
    ssems = (ssem0, ssem1)

    def load_round(j, b):
        base = (wid + j * NW) * EC
        pltpu.sync_copy(src.at[pl.ds(base, EC)], srcs[b])
        pltpu.sync_copy(dst.at[pl.ds(base, EC)], dsts[b])
        return pltpu.async_copy(yh.at[srcs[b]], rows[b], gsems[b])

    gd = [None, None]
    sd = [None, None]
    gd[0] = load_round(0, 0)
    for j in range(EFULL):
        b = j % 2
        nb = (j + 1) % 2
        if j + 1 < EFULL:
            if sd[nb] is not None:
                sd[nb].wait()
            gd[nb] = load_round(j + 1, nb)
        gd[b].wait()
        sd[b] = pltpu.async_copy(rows[b], shared.at[dsts[b]], ssems[b],
                                 add=True)
    if sd[(EFULL - 2) % 2] is not None:
        sd[(EFULL - 2) % 2].wait()
    sd[(EFULL - 1) % 2].wait()

    @pl.when(wid < EREM)
    def _():
        base = (wid + EFULL * NW) * EC
        pltpu.sync_copy(src.at[pl.ds(base, EC)], src_v0)
        pltpu.sync_copy(dst.at[pl.ds(base, EC)], dst_v0)
        pltpu.async_copy(yh.at[src_v0], rows_v0, gsem0).wait()
        pltpu.sync_copy(rows_v0, shared.at[dst_v0], add=True)

    plsc.subcore_barrier()
    r0 = s * RB
    pltpu.sync_copy(shared.at[pl.ds(r0, RB)], acc_out.at[c, pl.ds(r0, RB)])

    @pl.when(s == NS - 1)
    def _():
        pltpu.sync_copy(shared.at[pl.ds(NS * RB, RTAIL)],
                        acc_out.at[c, pl.ds(NS * RB, RTAIL)])


@functools.cache
def _scatter_ea():
    return pl.kernel(
        _b_body_ea,
        out_type=jax.ShapeDtypeStruct((NC, N, DH), jnp.float32),
        mesh=_mesh(),
        scratch_types=[
            pltpu.VMEM((EC,), jnp.int32),
            pltpu.VMEM((EC, DH), jnp.float32),
            pltpu.VMEM_SHARED((N, DH), jnp.float32),
        ],
    )


@functools.cache
def _scatter():
    return pl.kernel(
        _b_body,
        out_type=jax.ShapeDtypeStruct((NC, N, DH), jnp.float32),
        mesh=_mesh(),
        scratch_types=[
            pltpu.VMEM((EC,), jnp.int32),
            pltpu.VMEM((EC,), jnp.int32),
            pltpu.VMEM((EC, DH), jnp.float32),
            pltpu.VMEM((EC,), jnp.int32),
            pltpu.VMEM((EC,), jnp.int32),
            pltpu.VMEM((EC, DH), jnp.float32),
            pltpu.VMEM_SHARED((N, DH), jnp.float32),
            pltpu.SemaphoreType.DMA,
            pltpu.SemaphoreType.DMA,
            pltpu.SemaphoreType.DMA,
            pltpu.SemaphoreType.DMA,
        ],
    )


def _c_body(t_ref, a0_ref, a1_ref, eaa_ref, wea_ref, al_ref, f_ref):
    a = al_ref[0, :]
    a = a - jnp.max(a)
    e = jnp.exp(a)
    a = e / jnp.sum(e)
    wea = a[0] * wea_ref[0]
    for k in range(1, K):
        wea = wea + a[k] * wea_ref[k]
    agg = jnp.concatenate([a0_ref[0] + a0_ref[1], a1_ref[0] + a1_ref[1]], axis=1)
    eat = (eaa_ref[0] + eaa_ref[1])[:, :DE]
    f_ref[...] = (t_ref[...] + agg
                  + jnp.dot(eat, wea, preferred_element_type=jnp.float32))


def _run_c(t, acc0, acc1, ea_acc, wea_k, alpha):
    blk = 1000
    grid = N // blk
    return pl.pallas_call(
        _c_body,
        grid=(grid,),
        in_specs=[
            pl.BlockSpec((blk, D), lambda i: (i, 0)),
            pl.BlockSpec((NC, blk, DH), lambda i: (0, i, 0)),
            pl.BlockSpec((NC, blk, DH), lambda i: (0, i, 0)),
            pl.BlockSpec((NC, blk, DH), lambda i: (0, i, 0)),
            pl.BlockSpec((K, DE, D), lambda i: (0, 0, 0)),
            pl.BlockSpec((1, K), lambda i: (0, 0)),
        ],
        out_specs=pl.BlockSpec((blk, D), lambda i: (i, 0)),
        out_shape=jax.ShapeDtypeStruct((N, D), jnp.float32),
    )(t, acc0, acc1, ea_acc, wea_k, alpha)


def _d_body(ph_ref, pt_ref, idx_ref, wn_ref):
    px = pt_ref[0:1, :]
    py = pt_ref[1:2, :]
    pz = pt_ref[2:3, :]
    dx = ph_ref[:, 0:1] - px
    dy = ph_ref[:, 1:2] - py
    dz = ph_ref[:, 2:3] - pz
    d2 = (dx * dx + dy * dy) + dz * dz
    cid = lax.broadcasted_iota(jnp.int32, d2.shape, 1)
    ids = []
    ws = []
    for _ in range(KNN):
        m = jnp.min(d2, axis=1, keepdims=True)
        am = jnp.min(jnp.where(d2 == m, cid, N), axis=1, keepdims=True)
        ids.append(am)
        ws.append(1.0 / (m + 1e-16))
        d2 = jnp.where(cid == am, 3.0e38, d2)
    w = jnp.concatenate(ws, axis=1)
    idx_ref[...] = jnp.concatenate(ids, axis=1)
    wn_ref[...] = w / jnp.sum(w, axis=1, keepdims=True)


def _run_d(pos_high, pos_t):
    blk = 200
    grid = NH // blk
    return pl.pallas_call(
        _d_body,
        grid=(grid,),
        in_specs=[
            pl.BlockSpec((blk, 3), lambda i: (i, 0)),
            pl.BlockSpec((3, N), lambda i: (0, 0)),
        ],
        out_specs=[
            pl.BlockSpec((blk, KNN), lambda i: (i, 0)),
            pl.BlockSpec((blk, KNN), lambda i: (i, 0)),
        ],
        out_shape=[
            jax.ShapeDtypeStruct((NH, KNN), jnp.int32),
            jax.ShapeDtypeStruct((NH, KNN), jnp.float32),
        ],
    )(pos_high, pos_t)


def _e_body(f, i0, i1, i2, g0, g1, g2,
            idx_v0, rows_v0, idx_v1, rows_v1, gsem0, gsem1, ssem0, ssem1):
    c = lax.axis_index("c")
    s = lax.axis_index("s")
    wid = s * NC + c
    idxs = (idx_v0, idx_v1)
    rows = (rows_v0, rows_v1)
    gsems = (gsem0, gsem1)
    ssems = (ssem0, ssem1)

    rounds = []
    for ih, gh in ((i0, g0), (i1, g1), (i2, g2)):
        for j in range(GN_CH):
            rounds.append((ih, gh, wid * G_PER_W + j * GC))
    n = len(rounds)

    def load_round(k, b):
        ih, _, base = rounds[k]
        pltpu.sync_copy(ih.at[pl.ds(base, GC)], idxs[b])
        return pltpu.async_copy(f.at[idxs[b]], rows[b], gsems[b])

    gd = [None, None]
    sd = [None, None]
    gd[0] = load_round(0, 0)
    for k in range(n):
        b = k % 2
        nb = (k + 1) % 2
        if k + 1 < n:
            if sd[nb] is not None:
                sd[nb].wait()
            gd[nb] = load_round(k + 1, nb)
        gd[b].wait()
        _, gh, base = rounds[k]
        sd[b] = pltpu.async_copy(rows[b], gh.at[pl.ds(base, GC)], ssems[b])
    sd[(n - 2) % 2].wait()
    sd[(n - 1) % 2].wait()


@functools.cache
def _gather3():
    return pl.kernel(
        _e_body,
        out_type=(
            jax.ShapeDtypeStruct((NHP, D), jnp.float32),
            jax.ShapeDtypeStruct((NHP, D), jnp.float32),
            jax.ShapeDtypeStruct((NHP, D), jnp.float32),
        ),
        mesh=_mesh(),
        scratch_types=[
            pltpu.VMEM((GC,), jnp.int32),
            pltpu.VMEM((GC, D), jnp.float32),
            pltpu.VMEM((GC,), jnp.int32),
            pltpu.VMEM((GC, D), jnp.float32),
            pltpu.SemaphoreType.DMA,
            pltpu.SemaphoreType.DMA,
            pltpu.SemaphoreType.DMA,
            pltpu.SemaphoreType.DMA,
        ],
    )


def _f_body(g0_ref, g1_ref, g2_ref, wn_ref, o_ref):
    wn = wn_ref[...]
    o_ref[...] = (g0_ref[...] * wn[:, 0:1] + g1_ref[...] * wn[:, 1:2]
                  + g2_ref[...] * wn[:, 2:3])


def _run_f(g0, g1, g2, wn):
    blk = 200
    grid = NH // blk
    return pl.pallas_call(
        _f_body,
        grid=(grid,),
        in_specs=[
            pl.BlockSpec((blk, D), lambda i: (i, 0)),
            pl.BlockSpec((blk, D), lambda i: (i, 0)),
            pl.BlockSpec((blk, D), lambda i: (i, 0)),
            pl.BlockSpec((blk, KNN), lambda i: (i, 0)),
        ],
        out_specs=pl.BlockSpec((blk, D), lambda i: (i, 0)),
        out_shape=jax.ShapeDtypeStruct((NH, D), jnp.float32),
    )(g0, g1, g2, wn)


@jax.jit
def _kernel_impl(x, edge_index, edge_attr, pos, pos_high, W2, alpha2, Wr2, b2):
    wxx_k = W2[:, :D, :]
    wea_k = W2[:, D:, :]
    alpha = alpha2.reshape(1, K)
    brow = b2.reshape(1, D)
    src = edge_index[0]
    dst = edge_index[1]
    eap = jnp.pad(edge_attr, ((0, 0), (0, DH - DE)))
    z128 = jnp.zeros((N, DH), jnp.float32)

    y0, y1, t = _run_a(x, wxx_k, alpha, Wr2, brow)
    scatter = _scatter()
    acc0 = scatter(y0, src, dst, z128)
    acc1 = scatter(y1, src, dst, z128)
    ea_acc = _scatter_ea()(eap, dst, z128)
    f = _run_c(t, acc0, acc1, ea_acc, wea_k, alpha)

    pos_t = pos.T
    idx, wn = _run_d(pos_high, pos_t)
    ipad = jnp.zeros((KNN, NHP), jnp.int32).at[:, :NH].set(idx.T)
    g0, g1, g2 = _gather3()(f, ipad[0], ipad[1], ipad[2])
    return _run_f(g0, g1, g2, wn)


def kernel(x, edge_index, edge_attr, pos, pos_high, W1, alpha1, Wr1, b1,
           W2, alpha2, Wr2, b2):
    return _kernel_impl(x, edge_index, edge_attr, pos, pos_high,
                        W2, alpha2, Wr2, b2)
